# baseline (speedup 1.0000x reference)
import functools
import jax
import jax.numpy as jnp
from jax import lax
from jax.experimental import pallas as pl
from jax.experimental.pallas import tpu as pltpu
from jax.experimental.pallas import tpu_sc as plsc

X_LEN = 50
D = 32
F = 26
FL = F * X_LEN
FLD = FL * D
HALF = FLD // 2
FLH = FL // 2
B = 1024
NW = 32
IPW = B // NW
L = 16
SROW = 17


def _rsqrt16(v):
    i = plsc.bitcast(v, jnp.int32)
    i = jnp.int32(0x5F3759DF) - (i >> 1)
    y = plsc.bitcast(i, jnp.float32)
    for _ in range(3):
        y = y * (1.5 - 0.5 * v * y * y)
    return y


def sc_call(x2, tdl, gamma, beta_eff):
    mesh = plsc.VectorSubcoreMesh(core_axis_name="c", subcore_axis_name="s")

    @functools.partial(
        pl.kernel,
        mesh=mesh,
        compiler_params=pltpu.CompilerParams(
            needs_layout_passes=False, use_tc_tiling_on_sc=False),
        out_type=jax.ShapeDtypeStruct((B, FLD), jnp.float32),
        scratch_types=[
            pltpu.VMEM((FLD,), jnp.float32),
            pltpu.VMEM((FLD,), jnp.float32),
            pltpu.VMEM((HALF,), jnp.float32),
            pltpu.VMEM((HALF,), jnp.float32),
            pltpu.VMEM((D * X_LEN,), jnp.float32),
            pltpu.VMEM((D * SROW,), jnp.float32),
            pltpu.VMEM((D * SROW,), jnp.float32),
            pltpu.VMEM((D,), jnp.float32),
            pltpu.VMEM((D,), jnp.float32),
            pltpu.SemaphoreType.DMA,
            pltpu.SemaphoreType.DMA,
            pltpu.SemaphoreType.DMA,
            pltpu.SemaphoreType.DMA,
        ],
    )
    def k(x_hbm, t_hbm, g_hbm, b_hbm, out_hbm,
          inA, inB, oh0, oh1, tbl, s17a, s17b, gv, bv, siA, siB, so0, so1):
        inb = (inA, inB)
        sin = (siA, siB)
        outh = (oh0, oh1)
        sout = (so0, so1)
        stash = (s17a, s17b)
        wid = lax.axis_index("s") * 2 + lax.axis_index("c")
        pltpu.sync_copy(t_hbm, tbl)
        pltpu.sync_copy(g_hbm, gv)
        pltpu.sync_copy(b_hbm, bv)
        iota = lax.iota(jnp.int32, L)
        i17 = iota * SROW
        gvec = (gv[pl.ds(0, L)], gv[pl.ds(L, L)])
        bvec = (bv[pl.ds(0, L)], bv[pl.ds(L, L)])

        def in_copy(q, item, dh):
            return pltpu.make_async_copy(
                x_hbm.at[item, pl.ds(dh * HALF, HALF)],
                inb[q].at[pl.ds(dh * HALF, HALF)], sin[q])

        def out_copy(h, item):
            return pltpu.make_async_copy(
                outh[h], out_hbm.at[item, pl.ds(h * HALF, HALF)], sout[h])

        def do_chunk(q, h, start, s17):
            gfl = h * FLH
            lvec = lax.rem(iota + (gfl % X_LEN + start), X_LEN)
            s = None
            s2 = None
            for d in range(D):
                v = inb[q][pl.ds(d * FL + gfl + start, L)]
                t = plsc.load_gather(tbl, [lvec + d * X_LEN])
                w = v + t
                s17[pl.ds(d * SROW, L)] = w
                s = w if s is None else s + w
                s2 = w * w if s2 is None else s2 + w * w
            mean = s * (1.0 / D)
            var = s2 * (1.0 / D) - mean * mean
            rs = _rsqrt16(var + 1e-5)
            for flc in range(L):
                mean_f = mean[flc]
                rs_f = rs[flc]
                g0 = plsc.load_gather(s17, [i17 + flc])
                g1 = plsc.load_gather(s17, [i17 + (L * SROW + flc)])
                o0 = (g0 - mean_f) * rs_f * gvec[0] + bvec[0]
                o1 = (g1 - mean_f) * rs_f * gvec[1] + bvec[1]
                obase = (start + flc) * D
                outh[h][pl.ds(obase, L)] = o0
                outh[h][pl.ds(obase + L, L)] = o1

        def compute_half(q, h):
            npair = FLH // (2 * L)
            def pair_body(c, carry2):
                base = c * (2 * L)
                do_chunk(q, h, base, stash[0])
                do_chunk(q, h, base + L, stash[1])
                return carry2
            lax.fori_loop(0, npair, pair_body, 0)
            do_chunk(q, h, FLH - L, stash[0])

        b0 = wid * IPW
        in_copy(0, b0, 0).start()
        in_copy(0, b0, 1).start()
        in_copy(1, b0 + 1, 0).start()
        in_copy(1, b0 + 1, 1).start()

        def item_pair(g, carry):
            for q in range(2):
                it = 2 * g + q
                b = b0 + it
                in_copy(q, b0, 0).wait()
                in_copy(q, b0, 1).wait()
                for h in range(2):
                    @pl.when(it > 0)
                    def _():
                        out_copy(h, b0).wait()

                    compute_half(q, h)
                    out_copy(h, b).start()

                @pl.when(it + 2 < IPW)
                def _():
                    in_copy(q, b + 2, 0).start()
                    in_copy(q, b + 2, 1).start()

            return carry

        lax.fori_loop(0, IPW // 2, item_pair, 0)
        out_copy(0, b0).wait()
        out_copy(1, b0).wait()

    return k(x2, tdl.reshape(-1), gamma, beta_eff)


def kernel(x, table, gamma, beta, batch_size):
    batch = x.shape[0]
    resid = (jnp.asarray(batch_size, jnp.int32) - batch).astype(jnp.float32)
    beta_eff = beta + resid
    x2 = x.reshape(batch, FLD)
    tdl = table.T
    out = sc_call(x2, tdl, gamma, beta_eff)
    return out.reshape(batch, F, X_LEN, D)


# speedup vs baseline: 1.0513x; 1.0513x over previous
import functools
import jax
import jax.numpy as jnp
from jax import lax
from jax.experimental import pallas as pl
from jax.experimental.pallas import tpu as pltpu
from jax.experimental.pallas import tpu_sc as plsc

X_LEN = 50
D = 32
F = 26
FL = F * X_LEN
FLD = FL * D
B = 1024
NW = 32
IPW = B // NW
L = 16
SZ = (648, 652)
OFF = (0, 648)
SROW = 17


def _rsqrt16(v):
    i = plsc.bitcast(v, jnp.int32)
    i = jnp.int32(0x5F3759DF) - (i >> 1)
    y = plsc.bitcast(i, jnp.float32)
    for _ in range(3):
        y = y * (1.5 - 0.5 * v * y * y)
    return y


def sc_call(x3, tfl, gamma, beta_eff):
    mesh = plsc.VectorSubcoreMesh(core_axis_name="c", subcore_axis_name="s")

    @functools.partial(
        pl.kernel,
        mesh=mesh,
        compiler_params=pltpu.CompilerParams(
            needs_layout_passes=False, use_tc_tiling_on_sc=False),
        out_type=jax.ShapeDtypeStruct((B, FLD), jnp.float32),
        scratch_types=[
            pltpu.VMEM((D, SZ[0]), jnp.float32),
            pltpu.VMEM((D, SZ[1]), jnp.float32),
            pltpu.VMEM((SZ[0] * D,), jnp.float32),
            pltpu.VMEM((SZ[1] * D,), jnp.float32),
            pltpu.VMEM((FLD,), jnp.float32),
            pltpu.VMEM((D * SROW,), jnp.float32),
            pltpu.VMEM((D * SROW,), jnp.float32),
            pltpu.VMEM((D,), jnp.float32),
            pltpu.VMEM((D,), jnp.float32),
            pltpu.SemaphoreType.DMA,
            pltpu.SemaphoreType.DMA,
            pltpu.SemaphoreType.DMA,
            pltpu.SemaphoreType.DMA,
        ],
    )
    def k(x_hbm, t_hbm, g_hbm, b_hbm, out_hbm,
          in0, in1, o0, o1, tT, s17a, s17b, gv, bv, si0, si1, so0, so1):
        inb = (in0, in1)
        outb = (o0, o1)
        sin = (si0, si1)
        sout = (so0, so1)
        stash = (s17a, s17b)
        wid = lax.axis_index("s") * 2 + lax.axis_index("c")
        pltpu.sync_copy(t_hbm, tT)
        pltpu.sync_copy(g_hbm, gv)
        pltpu.sync_copy(b_hbm, bv)
        iota = lax.iota(jnp.int32, L)
        i17 = iota * SROW
        gvec = (gv[pl.ds(0, L)], gv[pl.ds(L, L)])
        bvec = (bv[pl.ds(0, L)], bv[pl.ds(L, L)])

        def in_copy(p, item):
            return pltpu.make_async_copy(
                x_hbm.at[item, :, pl.ds(OFF[p], SZ[p])], inb[p], sin[p])

        def out_copy(p, item):
            return pltpu.make_async_copy(
                outb[p], out_hbm.at[item, pl.ds(OFF[p] * D, SZ[p] * D)], sout[p])

        def do_chunk(p, start, s17):
            goff = OFF[p]
            s = None
            s2 = None
            for d in range(D):
                v = inb[p][d, pl.ds(start, L)]
                t = tT[pl.ds(d * FL + goff + start, L)]
                w = v + t
                s17[pl.ds(d * SROW, L)] = w
                s = w if s is None else s + w
                s2 = w * w if s2 is None else s2 + w * w
            mean = s * (1.0 / D)
            var = s2 * (1.0 / D) - mean * mean
            rs = _rsqrt16(var + 1e-5)
            for flc in range(L):
                mean_f = mean[flc]
                rs_f = rs[flc]
                g0 = plsc.load_gather(s17, [i17 + flc])
                g1 = plsc.load_gather(s17, [i17 + (L * SROW + flc)])
                o0 = (g0 - mean_f) * rs_f * gvec[0] + bvec[0]
                o1 = (g1 - mean_f) * rs_f * gvec[1] + bvec[1]
                obase = (start + flc) * D
                outb[p][pl.ds(obase, L)] = o0
                outb[p][pl.ds(obase + L, L)] = o1

        def compute(p):
            szc = SZ[p]
            npair = szc // (2 * L)
            def pair_body(c, carry2):
                base = c * (2 * L)
                do_chunk(p, base, stash[0])
                do_chunk(p, base + L, stash[1])
                return carry2
            lax.fori_loop(0, npair, pair_body, 0)
            do_chunk(p, szc - L, stash[0])

        b0 = wid * IPW
        in_copy(0, b0).start()
        in_copy(1, b0).start()

        def item_body(it, carry):
            b = b0 + it
            for p in range(2):
                in_copy(p, b0).wait()

                @pl.when(it > 0)
                def _():
                    out_copy(p, b0).wait()

                compute(p)
                out_copy(p, b).start()

                @pl.when(it + 1 < IPW)
                def _():
                    in_copy(p, b + 1).start()

            return carry

        lax.fori_loop(0, IPW, item_body, 0)
        out_copy(0, b0).wait()
        out_copy(1, b0).wait()

    return k(x3, tfl.reshape(-1), gamma, beta_eff)


def kernel(x, table, gamma, beta, batch_size):
    batch = x.shape[0]
    resid = (jnp.asarray(batch_size, jnp.int32) - batch).astype(jnp.float32)
    beta_eff = beta + resid
    x3 = x.reshape(batch, D, FL)
    tfl = jnp.tile(table.T[:, None, :], (1, F, 1)).reshape(D, FL)
    out = sc_call(x3, tfl, gamma, beta_eff)
    return out.reshape(batch, F, X_LEN, D)
